# baseline (speedup 1.0000x reference)
import functools

import jax
import jax.numpy as jnp
from jax import lax
from jax.experimental import pallas as pl
from jax.experimental.pallas import tpu as pltpu
from jax.experimental.pallas import tpu_sc as plsc

H = 16
N = 257
NREL = 964
B = 32
TABW = NREL * H
RPW = 8
NV = 16

RB = 8
NIB = (N + RB - 1) // RB


def _gather_rows(tab_v, idx_v, outb_v, r):
    def body(c, carry):
        cs = c * 16
        addr = idx_v[r, pl.ds(cs, 16)] * H
        for h in range(H):
            outb_v[h, r, pl.ds(cs, 16)] = plsc.load_gather(tab_v, [addr + h])
        return carry
    lax.fori_loop(0, NV, body, 0)
    lanes = lax.iota(jnp.int32, 16)
    rv = jnp.full((16,), r, jnp.int32)
    cv = jnp.full((16,), N - 1, jnp.int32)
    iv = plsc.load_gather(idx_v, [rv, cv])
    vals = plsc.load_gather(tab_v, [iv * H + lanes])
    plsc.store_scatter(outb_v, [lanes, rv, cv], vals)


def _sc_gather_body(tab_hbm, idx_hbm, out_hbm, tab_v, idx_v, outb_v):
    wid = lax.axis_index("s") * 2 + lax.axis_index("c")
    row0 = wid * RPW
    pltpu.sync_copy(tab_hbm, tab_v)
    pltpu.sync_copy(idx_hbm.at[pl.ds(row0, RPW), :], idx_v.at[:RPW])

    def body(r, carry):
        _gather_rows(tab_v, idx_v, outb_v, r)
        return carry
    lax.fori_loop(0, RPW, body, 0)
    pltpu.sync_copy(outb_v.at[:, :RPW], out_hbm.at[:, pl.ds(row0, RPW), :])

    @pl.when(wid == NW - 1)
    def _tail_row():
        pltpu.sync_copy(idx_hbm.at[pl.ds(N - 1, 1), :], idx_v.at[RPW:])
        _gather_rows(tab_v, idx_v, outb_v, RPW)
        pltpu.sync_copy(outb_v.at[:, RPW:], out_hbm.at[:, pl.ds(N - 1, 1), :])


NW = 32

_sc_gather = functools.partial(
    pl.kernel,
    out_type=jax.ShapeDtypeStruct((H, N, N), jnp.float32),
    mesh=plsc.VectorSubcoreMesh(core_axis_name="c", subcore_axis_name="s",
                                num_cores=2, num_subcores=16),
    compiler_params=pltpu.CompilerParams(needs_layout_passes=False),
    scratch_types=[
        pltpu.VMEM((TABW,), jnp.float32),
        pltpu.VMEM((RPW + 1, N), jnp.int32),
        pltpu.VMEM((H, RPW + 1, N), jnp.float32),
    ],
)(_sc_gather_body)


NBUF = 4
NCH = 2
CH = H // NCH
C = B * NCH


def _chunk_src(attn_hbm, c):
    return attn_hbm.at[c // NCH, pl.ds((c % NCH) * CH, CH)]


def _chunk_dst(out_hbm, c):
    return out_hbm.at[c // NCH, pl.ds((c % NCH) * CH, CH)]


def _add_manual(bias_hbm, attn_hbm, out_hbm, bias_v, abuf, obuf, bsem, isem, osem):
    c = pl.program_id(0)

    @pl.when(c == 0)
    def _prologue():
        for p in range(NBUF - 1):
            pltpu.async_copy(_chunk_src(attn_hbm, p), abuf.at[p], isem.at[p])
        pltpu.async_copy(bias_hbm, bias_v, bsem).wait()

    def _step(k):
        @pl.when(c + NBUF - 1 < C)
        def _prefetch():
            pltpu.async_copy(_chunk_src(attn_hbm, c + NBUF - 1),
                             abuf.at[(c + NBUF - 1) % NBUF],
                             isem.at[(c + NBUF - 1) % NBUF])

        pltpu.make_async_copy(_chunk_src(attn_hbm, c), abuf.at[k],
                              isem.at[k]).wait()

        @pl.when(c >= NBUF)
        def _wait_prev_out():
            pltpu.make_async_copy(obuf.at[k], _chunk_dst(out_hbm, c - NBUF),
                                  osem.at[k]).wait()

        h0 = (c % NCH) * CH
        obuf[k] = abuf[k] + bias_v[pl.ds(h0, CH)]
        pltpu.async_copy(obuf.at[k], _chunk_dst(out_hbm, c), osem.at[k])

    for k in range(NBUF):
        pl.when(c % NBUF == k)(lambda k=k: _step(k))

    @pl.when(c == C - 1)
    def _drain():
        for k in range(NBUF):
            cc = C - NBUF + ((k - C) % NBUF)
            pltpu.make_async_copy(obuf.at[k], _chunk_dst(out_hbm, cc),
                                  osem.at[k]).wait()


def kernel(attn, rel_pos_bias_table, rel_pos_index):
    tab_flat = rel_pos_bias_table.reshape(-1)
    bias3 = _sc_gather(tab_flat, rel_pos_index)

    return pl.pallas_call(
        _add_manual,
        grid=(C,),
        in_specs=[
            pl.BlockSpec(memory_space=pltpu.HBM),
            pl.BlockSpec(memory_space=pltpu.HBM),
        ],
        out_specs=pl.BlockSpec(memory_space=pltpu.HBM),
        out_shape=jax.ShapeDtypeStruct(attn.shape, attn.dtype),
        scratch_shapes=[
            pltpu.VMEM((H, N, N), jnp.float32),
            pltpu.VMEM((NBUF, CH, N, N), jnp.float32),
            pltpu.VMEM((NBUF, CH, N, N), jnp.float32),
            pltpu.SemaphoreType.DMA,
            pltpu.SemaphoreType.DMA((NBUF,)),
            pltpu.SemaphoreType.DMA((NBUF,)),
        ],
        compiler_params=pltpu.CompilerParams(vmem_limit_bytes=100 * 1024 * 1024),
    )(bias3, attn)


# speedup vs baseline: 1.0017x; 1.0017x over previous
import functools

import jax
import jax.numpy as jnp
from jax import lax
from jax.experimental import pallas as pl
from jax.experimental.pallas import tpu as pltpu
from jax.experimental.pallas import tpu_sc as plsc

H = 16
N = 257
NREL = 964
B = 32
TABW = NREL * H
RPW = 8
NV = 16

RB = 8
NIB = (N + RB - 1) // RB


def _gather_rows(tab_v, idx_v, outb_v, r):
    def body(c, carry):
        cs = c * 16
        addr = idx_v[r, pl.ds(cs, 16)] * H
        for h in range(H):
            outb_v[h, r, pl.ds(cs, 16)] = plsc.load_gather(tab_v, [addr + h])
        return carry
    lax.fori_loop(0, NV, body, 0)
    lanes = lax.iota(jnp.int32, 16)
    rv = jnp.full((16,), r, jnp.int32)
    cv = jnp.full((16,), N - 1, jnp.int32)
    iv = plsc.load_gather(idx_v, [rv, cv])
    vals = plsc.load_gather(tab_v, [iv * H + lanes])
    plsc.store_scatter(outb_v, [lanes, rv, cv], vals)


def _sc_gather_body(tab_hbm, idx_hbm, out_hbm, tab_v, idx_v, outb_v):
    wid = lax.axis_index("s") * 2 + lax.axis_index("c")
    row0 = wid * RPW
    pltpu.sync_copy(tab_hbm, tab_v)
    pltpu.sync_copy(idx_hbm.at[pl.ds(row0, RPW), :], idx_v.at[:RPW])

    def body(r, carry):
        _gather_rows(tab_v, idx_v, outb_v, r)
        return carry
    lax.fori_loop(0, RPW, body, 0)
    pltpu.sync_copy(outb_v.at[:, :RPW], out_hbm.at[:, pl.ds(row0, RPW), :])

    @pl.when(wid == NW - 1)
    def _tail_row():
        pltpu.sync_copy(idx_hbm.at[pl.ds(N - 1, 1), :], idx_v.at[RPW:])
        _gather_rows(tab_v, idx_v, outb_v, RPW)
        pltpu.sync_copy(outb_v.at[:, RPW:], out_hbm.at[:, pl.ds(N - 1, 1), :])


NW = 32

_sc_gather = functools.partial(
    pl.kernel,
    out_type=jax.ShapeDtypeStruct((H, N, N), jnp.float32),
    mesh=plsc.VectorSubcoreMesh(core_axis_name="c", subcore_axis_name="s",
                                num_cores=2, num_subcores=16),
    compiler_params=pltpu.CompilerParams(needs_layout_passes=False),
    scratch_types=[
        pltpu.VMEM((TABW,), jnp.float32),
        pltpu.VMEM((RPW + 1, N), jnp.int32),
        pltpu.VMEM((H, RPW + 1, N), jnp.float32),
    ],
)(_sc_gather_body)


NBUF = 4
NCH = 2
CH = H // NCH
C = B * NCH


def _chunk_src(attn_hbm, c):
    return attn_hbm.at[c // NCH, pl.ds((c % NCH) * CH, CH)]


def _chunk_dst(out_hbm, c):
    return out_hbm.at[c // NCH, pl.ds((c % NCH) * CH, CH)]


def _add_manual(bias_hbm, attn_hbm, out_hbm, bias_v, abuf, obuf, bsem, isem, osem):
    c = pl.program_id(0)

    @pl.when(c == 0)
    def _prologue():
        for p in range(NBUF - 1):
            pltpu.async_copy(_chunk_src(attn_hbm, p), abuf.at[p], isem.at[p])
        pltpu.async_copy(bias_hbm, bias_v, bsem).wait()

    def _step(k):
        @pl.when(c + NBUF - 1 < C)
        def _prefetch():
            kk = (NBUF - 1 + k) % NBUF
            pltpu.async_copy(_chunk_src(attn_hbm, c + NBUF - 1),
                             abuf.at[kk], isem.at[kk])

        pltpu.make_async_copy(_chunk_src(attn_hbm, c), abuf.at[k],
                              isem.at[k]).wait()

        @pl.when(c >= NBUF)
        def _wait_prev_out():
            pltpu.make_async_copy(obuf.at[k], _chunk_dst(out_hbm, c - NBUF),
                                  osem.at[k]).wait()

        h0 = (c % NCH) * CH
        obuf[k] = abuf[k] + bias_v[pl.ds(h0, CH)]
        pltpu.async_copy(obuf.at[k], _chunk_dst(out_hbm, c), osem.at[k],
                         priority=1)

    for k in range(NBUF):
        pl.when(c % NBUF == k)(lambda k=k: _step(k))

    @pl.when(c == C - 1)
    def _drain():
        for k in range(NBUF):
            cc = C - NBUF + ((k - C) % NBUF)
            pltpu.make_async_copy(obuf.at[k], _chunk_dst(out_hbm, cc),
                                  osem.at[k]).wait()


def kernel(attn, rel_pos_bias_table, rel_pos_index):
    tab_flat = rel_pos_bias_table.reshape(-1)
    bias3 = _sc_gather(tab_flat, rel_pos_index)

    return pl.pallas_call(
        _add_manual,
        grid=(C,),
        in_specs=[
            pl.BlockSpec(memory_space=pltpu.HBM),
            pl.BlockSpec(memory_space=pltpu.HBM),
        ],
        out_specs=pl.BlockSpec(memory_space=pltpu.HBM),
        out_shape=jax.ShapeDtypeStruct(attn.shape, attn.dtype),
        scratch_shapes=[
            pltpu.VMEM((H, N, N), jnp.float32),
            pltpu.VMEM((NBUF, CH, N, N), jnp.float32),
            pltpu.VMEM((NBUF, CH, N, N), jnp.float32),
            pltpu.SemaphoreType.DMA,
            pltpu.SemaphoreType.DMA((NBUF,)),
            pltpu.SemaphoreType.DMA((NBUF,)),
        ],
        compiler_params=pltpu.CompilerParams(vmem_limit_bytes=100 * 1024 * 1024),
    )(bias3, attn)


# speedup vs baseline: 2.1206x; 2.1170x over previous
import functools

import jax
import jax.numpy as jnp
from jax import lax
from jax.experimental import pallas as pl
from jax.experimental.pallas import tpu as pltpu
from jax.experimental.pallas import tpu_sc as plsc

H = 16
N = 257
NREL = 964
B = 32
TABW = NREL * H
NW = 32
RPW = 8
NV = 16


def _gather_rows(tab_v, idx_v, outb_v, r):
    def body(c, carry):
        cs = c * 16
        addr = idx_v[r, pl.ds(cs, 16)] * H
        for h in range(H):
            outb_v[h, r, pl.ds(cs, 16)] = plsc.load_gather(tab_v, [addr + h])
        return carry
    lax.fori_loop(0, NV, body, 0)
    lanes = lax.iota(jnp.int32, 16)
    rv = jnp.full((16,), r, jnp.int32)
    cv = jnp.full((16,), N - 1, jnp.int32)
    iv = plsc.load_gather(idx_v, [rv, cv])
    vals = plsc.load_gather(tab_v, [iv * H + lanes])
    plsc.store_scatter(outb_v, [lanes, rv, cv], vals)


def _sc_gather_body(tab_hbm, idx_hbm, out_hbm, tab_v, idx_v, outb_v):
    wid = lax.axis_index("s") * 2 + lax.axis_index("c")
    row0 = wid * RPW
    pltpu.sync_copy(tab_hbm, tab_v)
    pltpu.sync_copy(idx_hbm.at[pl.ds(row0, RPW), :], idx_v.at[:RPW])

    def body(r, carry):
        _gather_rows(tab_v, idx_v, outb_v, r)
        return carry
    lax.fori_loop(0, RPW, body, 0)
    pltpu.sync_copy(outb_v.at[:, :RPW], out_hbm.at[:, pl.ds(row0, RPW), :])

    @pl.when(wid == NW - 1)
    def _tail_row():
        pltpu.sync_copy(idx_hbm.at[pl.ds(N - 1, 1), :], idx_v.at[RPW:])
        _gather_rows(tab_v, idx_v, outb_v, RPW)
        pltpu.sync_copy(outb_v.at[:, RPW:], out_hbm.at[:, pl.ds(N - 1, 1), :])


_sc_gather = functools.partial(
    pl.kernel,
    out_type=jax.ShapeDtypeStruct((H, N, N), jnp.float32),
    mesh=plsc.VectorSubcoreMesh(core_axis_name="c", subcore_axis_name="s",
                                num_cores=2, num_subcores=16),
    compiler_params=pltpu.CompilerParams(needs_layout_passes=False),
    scratch_types=[
        pltpu.VMEM((TABW,), jnp.float32),
        pltpu.VMEM((RPW + 1, N), jnp.int32),
        pltpu.VMEM((H, RPW + 1, N), jnp.float32),
    ],
)(_sc_gather_body)


def kernel(attn, rel_pos_bias_table, rel_pos_index):
    tab_flat = rel_pos_bias_table.reshape(-1)
    bias3 = _sc_gather(tab_flat, rel_pos_index)
    return attn + bias3[None]
